# Initial kernel scaffold; baseline (speedup 1.0000x reference)
import jax
import jax.numpy as jnp
from jax.experimental import pallas as pl


def kernel(x, edge_index, Wq, bq, Wk, bk, Wv, bv, Ow, Ob, ln1_g, ln1_b, ln2_g, ln2_b):
    raise NotImplementedError("write your pallas kernel here")


# speedup vs baseline: 28.0721x; 28.0721x over previous
import jax
import jax.numpy as jnp
from jax import lax
from jax.experimental import pallas as pl
from jax.experimental.pallas import tpu as pltpu
from jax.experimental.pallas import tpu_sc as plsc

N = 10000
E = 320000
D = 128
H = 8
DH = 16

NC = 2
NS = 16
NW = NC * NS
PER_W = E // NW
C = 80
ITERS = PER_W // C
NP = 10240
STRIPE = NP // NS

_f32 = jnp.float32


def _qkv_body(x_ref, wq_ref, bq_ref, wk_ref, bk_ref, wv_ref, bv_ref,
              q_ref, k_ref, v_ref):
    x = x_ref[...]
    q_ref[...] = jnp.dot(x, wq_ref[...], preferred_element_type=_f32) + bq_ref[...]
    k_ref[...] = jnp.dot(x, wk_ref[...], preferred_element_type=_f32) + bk_ref[...]
    v_ref[...] = jnp.dot(x, wv_ref[...], preferred_element_type=_f32) + bv_ref[...]


def _qkv(x, Wq, bq, Wk, bk, Wv, bv):
    bn = 1000
    grid = N // bn
    row_spec = pl.BlockSpec((bn, D), lambda i: (i, 0))
    w_spec = pl.BlockSpec((D, D), lambda i: (0, 0))
    b_spec = pl.BlockSpec((1, D), lambda i: (0, 0))
    return pl.pallas_call(
        _qkv_body,
        grid=(grid,),
        in_specs=[row_spec, w_spec, b_spec, w_spec, b_spec, w_spec, b_spec],
        out_specs=[row_spec, row_spec, row_spec],
        out_shape=[jax.ShapeDtypeStruct((N, D), _f32)] * 3,
    )(x, Wq, bq, Wk, bk, Wv, bv)


def _gather_body(q_hbm, k_hbm, v_hbm, src_hbm, dst_hbm,
                 ksrc_out, qdst_out, vsrc_out,
                 sidx, didx, kbuf, qbuf, vbuf, sem):
    wid = lax.axis_index("s") * NC + lax.axis_index("c")
    base = wid * PER_W

    def body(i, _):
        off = pl.multiple_of(base + i * C, 8)
        pltpu.sync_copy(src_hbm.at[pl.ds(off, C)], sidx)
        pltpu.sync_copy(dst_hbm.at[pl.ds(off, C)], didx)
        cpk = pltpu.async_copy(k_hbm.at[sidx], kbuf, sem)
        cpq = pltpu.async_copy(q_hbm.at[didx], qbuf, sem)
        cpv = pltpu.async_copy(v_hbm.at[sidx], vbuf, sem)
        cpk.wait()
        cpq.wait()
        cpv.wait()
        pltpu.sync_copy(kbuf, ksrc_out.at[pl.ds(off, C)])
        pltpu.sync_copy(qbuf, qdst_out.at[pl.ds(off, C)])
        pltpu.sync_copy(vbuf, vsrc_out.at[pl.ds(off, C)])
        return 0

    lax.fori_loop(0, ITERS, body, 0)


def _sc_gather(Q, K, V, src, dst):
    mesh = plsc.VectorSubcoreMesh(core_axis_name="c", subcore_axis_name="s")
    fn = pl.kernel(
        _gather_body,
        out_type=[jax.ShapeDtypeStruct((E, D), _f32)] * 3,
        mesh=mesh,
        scratch_types=[
            pltpu.VMEM((C,), jnp.int32),
            pltpu.VMEM((C,), jnp.int32),
            pltpu.VMEM((C, D), _f32),
            pltpu.VMEM((C, D), _f32),
            pltpu.VMEM((C, D), _f32),
            pltpu.SemaphoreType.DMA,
        ],
    )
    return fn(Q, K, V, src, dst)


def _msg_body(ksrc_ref, qdst_ref, vsrc_ref, msgv_ref, srep_ref):
    prod = ksrc_ref[...] * qdst_ref[...]
    jr = lax.broadcasted_iota(jnp.int32, (D, H), 0) // DH
    jc = lax.broadcasted_iota(jnp.int32, (D, H), 1)
    s8 = (jr == jc).astype(_f32)
    score = jnp.dot(prod, s8, preferred_element_type=_f32)
    score = jnp.exp(jnp.clip(score * 0.25, -5.0, 5.0))
    hr = lax.broadcasted_iota(jnp.int32, (H, D), 0)
    hc = lax.broadcasted_iota(jnp.int32, (H, D), 1) // DH
    s128 = (hr == hc).astype(_f32)
    srep = jnp.dot(score, s128, preferred_element_type=_f32)
    msgv_ref[...] = vsrc_ref[...] * srep
    srep_ref[...] = srep


def _msg(ksrc, qdst, vsrc):
    be = 2000
    grid = E // be
    row_spec = pl.BlockSpec((be, D), lambda i: (i, 0))
    return pl.pallas_call(
        _msg_body,
        grid=(grid,),
        in_specs=[row_spec, row_spec, row_spec],
        out_specs=[row_spec, row_spec],
        out_shape=[jax.ShapeDtypeStruct((E, D), _f32),
                   jax.ShapeDtypeStruct((E, D), _f32)],
    )(ksrc, qdst, vsrc)


def _scatter_body(msg_hbm, dst_hbm, zeros_hbm, out,
                  didx, buf, acc, sem):
    c = lax.axis_index("c")
    s = lax.axis_index("s")
    wid = s * NC + c
    r0 = pl.multiple_of(s * STRIPE, 8)

    pltpu.sync_copy(zeros_hbm.at[pl.ds(r0, STRIPE)], acc.at[pl.ds(r0, STRIPE)])
    plsc.subcore_barrier()

    base = wid * PER_W

    def body(i, _):
        off = pl.multiple_of(base + i * C, 8)
        pltpu.sync_copy(dst_hbm.at[pl.ds(off, C)], didx)
        pltpu.async_copy(msg_hbm.at[pl.ds(off, C)], buf, sem).wait()
        pltpu.sync_copy(buf, acc.at[didx], add=True)
        return 0

    lax.fori_loop(0, ITERS, body, 0)
    plsc.subcore_barrier()

    o0 = pl.multiple_of(c * NP + r0, 8)
    pltpu.sync_copy(acc.at[pl.ds(r0, STRIPE)], out.at[pl.ds(o0, STRIPE)])


def _sc_scatter(msg, dst, zeros):
    mesh = plsc.VectorSubcoreMesh(core_axis_name="c", subcore_axis_name="s")
    fn = pl.kernel(
        _scatter_body,
        out_type=[jax.ShapeDtypeStruct((NC * NP, D), _f32)],
        mesh=mesh,
        scratch_types=[
            pltpu.VMEM((C,), jnp.int32),
            pltpu.VMEM((C, D), _f32),
            pltpu.VMEM_SHARED((NP, D), _f32),
            pltpu.SemaphoreType.DMA,
        ],
    )
    (out,) = fn(msg, dst, zeros)
    return out


def _ln(h, g, b):
    m = jnp.mean(h, axis=-1, keepdims=True)
    v = jnp.mean((h - m) ** 2, axis=-1, keepdims=True)
    return (h - m) * lax.rsqrt(v + 1e-5) * g + b


def _final_body(pv0_ref, pv1_ref, pz0_ref, pz1_ref, x_ref,
                ow_ref, ob_ref, g1_ref, b1_ref, g2_ref, b2_ref, out_ref):
    wv = pv0_ref[...] + pv1_ref[...]
    zrep = pz0_ref[...] + pz1_ref[...]
    attn = wv / (zrep + 0.001)
    h1 = _ln(x_ref[...] + attn, g1_ref[...], b1_ref[...])
    h2 = _ln(h1, g2_ref[...], b2_ref[...])
    out_ref[...] = h1 + jax.nn.relu(
        jnp.dot(h2, ow_ref[...], preferred_element_type=_f32) + ob_ref[...])


def _final(outV, outZ, x, Ow, Ob, g1, b1, g2, b2):
    bn = 80
    grid = N // bn
    half = NP // bn
    row = pl.BlockSpec((bn, D), lambda i: (i, 0))
    row1 = pl.BlockSpec((bn, D), lambda i: (i + half, 0))
    w_spec = pl.BlockSpec((D, D), lambda i: (0, 0))
    b_spec = pl.BlockSpec((1, D), lambda i: (0, 0))
    return pl.pallas_call(
        _final_body,
        grid=(grid,),
        in_specs=[row, row1, row, row1, row,
                  w_spec, b_spec, b_spec, b_spec, b_spec, b_spec],
        out_specs=row,
        out_shape=jax.ShapeDtypeStruct((N, D), _f32),
    )(outV, outV, outZ, outZ, x, Ow, Ob, g1, b1, g2, b2)


def kernel(x, edge_index, Wq, bq, Wk, bk, Wv, bv, Ow, Ob,
           ln1_g, ln1_b, ln2_g, ln2_b):
    src = edge_index[0].astype(jnp.int32)
    dst = edge_index[1].astype(jnp.int32)
    bq2, bk2, bv2, ob2 = (b.reshape(1, D) for b in (bq, bk, bv, Ob))
    g1, b1, g2, b2 = (b.reshape(1, D) for b in (ln1_g, ln1_b, ln2_g, ln2_b))
    zeros = jnp.zeros((NP, D), _f32)

    Q, K, V = _qkv(x, Wq, bq2, Wk, bk2, Wv, bv2)
    ksrc, qdst, vsrc = _sc_gather(Q, K, V, src, dst)
    msgV, srep = _msg(ksrc, qdst, vsrc)
    outV = _sc_scatter(msgV, dst, zeros)
    outZ = _sc_scatter(srep, dst, zeros)
    return _final(outV, outZ, x, Ow, ob2, g1, b1, g2, b2)


# speedup vs baseline: 31.1465x; 1.1095x over previous
import jax
import jax.numpy as jnp
from jax import lax
from jax.experimental import pallas as pl
from jax.experimental.pallas import tpu as pltpu
from jax.experimental.pallas import tpu_sc as plsc

N = 10000
E = 320000
D = 128
H = 8
DH = 16

NC = 2
NS = 16
NW = NC * NS
PER_W = E // NW
C = 80
ITERS = PER_W // C
NP = 10240
STRIPE = NP // NS

_f32 = jnp.float32


def _qkv_body(x_ref, wq_ref, bq_ref, wk_ref, bk_ref, wv_ref, bv_ref,
              q_ref, kv_ref):
    x = x_ref[...]
    q = jnp.dot(x, wq_ref[...], preferred_element_type=_f32) + bq_ref[...]
    k = jnp.dot(x, wk_ref[...], preferred_element_type=_f32) + bk_ref[...]
    v = jnp.dot(x, wv_ref[...], preferred_element_type=_f32) + bv_ref[...]
    q_ref[...] = q
    k16 = lax.bitcast_convert_type(k.astype(jnp.bfloat16), jnp.uint16)
    v16 = lax.bitcast_convert_type(v.astype(jnp.bfloat16), jnp.uint16)
    kv_ref[...] = (v16.astype(jnp.int32) << 16) | k16.astype(jnp.int32)


def _qkv(x, Wq, bq, Wk, bk, Wv, bv):
    bn = 1000
    grid = N // bn
    row_spec = pl.BlockSpec((bn, D), lambda i: (i, 0))
    w_spec = pl.BlockSpec((D, D), lambda i: (0, 0))
    b_spec = pl.BlockSpec((1, D), lambda i: (0, 0))
    return pl.pallas_call(
        _qkv_body,
        grid=(grid,),
        in_specs=[row_spec, w_spec, b_spec, w_spec, b_spec, w_spec, b_spec],
        out_specs=[row_spec, row_spec],
        out_shape=[jax.ShapeDtypeStruct((N, D), _f32),
                   jax.ShapeDtypeStruct((N, D), jnp.int32)],
    )(x, Wq, bq, Wk, bk, Wv, bv)


def _gather_body(q_hbm, kv_hbm, src_hbm, dst_hbm,
                 kvsrc_out, qdst_out,
                 sidx, didx, kvbuf, qbuf, sem):
    wid = lax.axis_index("s") * NC + lax.axis_index("c")
    base = wid * PER_W

    def body(i, _):
        off = pl.multiple_of(base + i * C, 8)
        pltpu.sync_copy(src_hbm.at[pl.ds(off, C)], sidx)
        pltpu.sync_copy(dst_hbm.at[pl.ds(off, C)], didx)
        cpk = pltpu.async_copy(kv_hbm.at[sidx], kvbuf, sem)
        cpq = pltpu.async_copy(q_hbm.at[didx], qbuf, sem)
        cpk.wait()
        cpq.wait()
        pltpu.sync_copy(kvbuf, kvsrc_out.at[pl.ds(off, C)])
        pltpu.sync_copy(qbuf, qdst_out.at[pl.ds(off, C)])
        return 0

    lax.fori_loop(0, ITERS, body, 0)


def _sc_gather(Q, KV, src, dst):
    mesh = plsc.VectorSubcoreMesh(core_axis_name="c", subcore_axis_name="s")
    fn = pl.kernel(
        _gather_body,
        out_type=[jax.ShapeDtypeStruct((E, D), jnp.int32),
                  jax.ShapeDtypeStruct((E, D), _f32)],
        mesh=mesh,
        scratch_types=[
            pltpu.VMEM((C,), jnp.int32),
            pltpu.VMEM((C,), jnp.int32),
            pltpu.VMEM((C, D), jnp.int32),
            pltpu.VMEM((C, D), _f32),
            pltpu.SemaphoreType.DMA,
        ],
    )
    return fn(Q, KV, src, dst)


def _msg_body(kvsrc_ref, qdst_ref, msgv_ref, srep_ref):
    w = kvsrc_ref[...]
    ksrc = lax.bitcast_convert_type(w << 16, _f32)
    vsrc = lax.bitcast_convert_type(w & jnp.int32(-65536), _f32)
    prod = ksrc * qdst_ref[...]
    jr = lax.broadcasted_iota(jnp.int32, (D, H), 0) // DH
    jc = lax.broadcasted_iota(jnp.int32, (D, H), 1)
    s8 = (jr == jc).astype(_f32)
    score = jnp.dot(prod, s8, preferred_element_type=_f32)
    score = jnp.exp(jnp.clip(score * 0.25, -5.0, 5.0))
    hr = lax.broadcasted_iota(jnp.int32, (H, D), 0)
    hc = lax.broadcasted_iota(jnp.int32, (H, D), 1) // DH
    s128 = (hr == hc).astype(_f32)
    srep = jnp.dot(score, s128, preferred_element_type=_f32)
    msgv_ref[...] = vsrc * srep
    srep_ref[...] = srep


def _msg(kvsrc, qdst):
    be = 2000
    grid = E // be
    row_spec = pl.BlockSpec((be, D), lambda i: (i, 0))
    return pl.pallas_call(
        _msg_body,
        grid=(grid,),
        in_specs=[row_spec, row_spec],
        out_specs=[row_spec, row_spec],
        out_shape=[jax.ShapeDtypeStruct((E, D), _f32),
                   jax.ShapeDtypeStruct((E, D), _f32)],
    )(kvsrc, qdst)


def _scatter_body(msg_hbm, dst_hbm, zeros_hbm, out,
                  didx, buf, acc, sem):
    c = lax.axis_index("c")
    s = lax.axis_index("s")
    wid = s * NC + c
    r0 = pl.multiple_of(s * STRIPE, 8)

    pltpu.sync_copy(zeros_hbm.at[pl.ds(r0, STRIPE)], acc.at[pl.ds(r0, STRIPE)])
    plsc.subcore_barrier()

    base = wid * PER_W

    def body(i, _):
        off = pl.multiple_of(base + i * C, 8)
        pltpu.sync_copy(dst_hbm.at[pl.ds(off, C)], didx)
        pltpu.async_copy(msg_hbm.at[pl.ds(off, C)], buf, sem).wait()
        pltpu.sync_copy(buf, acc.at[didx], add=True)
        return 0

    lax.fori_loop(0, ITERS, body, 0)
    plsc.subcore_barrier()

    o0 = pl.multiple_of(c * NP + r0, 8)
    pltpu.sync_copy(acc.at[pl.ds(r0, STRIPE)], out.at[pl.ds(o0, STRIPE)])


def _sc_scatter(msg, dst, zeros):
    mesh = plsc.VectorSubcoreMesh(core_axis_name="c", subcore_axis_name="s")
    fn = pl.kernel(
        _scatter_body,
        out_type=[jax.ShapeDtypeStruct((NC * NP, D), _f32)],
        mesh=mesh,
        scratch_types=[
            pltpu.VMEM((C,), jnp.int32),
            pltpu.VMEM((C, D), _f32),
            pltpu.VMEM_SHARED((NP, D), _f32),
            pltpu.SemaphoreType.DMA,
        ],
    )
    (out,) = fn(msg, dst, zeros)
    return out


def _ln(h, g, b):
    m = jnp.mean(h, axis=-1, keepdims=True)
    v = jnp.mean((h - m) ** 2, axis=-1, keepdims=True)
    return (h - m) * lax.rsqrt(v + 1e-5) * g + b


def _final_body(pv0_ref, pv1_ref, pz0_ref, pz1_ref, x_ref,
                ow_ref, ob_ref, g1_ref, b1_ref, g2_ref, b2_ref, out_ref):
    wv = pv0_ref[...] + pv1_ref[...]
    zrep = pz0_ref[...] + pz1_ref[...]
    attn = wv / (zrep + 0.001)
    h1 = _ln(x_ref[...] + attn, g1_ref[...], b1_ref[...])
    h2 = _ln(h1, g2_ref[...], b2_ref[...])
    out_ref[...] = h1 + jax.nn.relu(
        jnp.dot(h2, ow_ref[...], preferred_element_type=_f32) + ob_ref[...])


def _final(outV, outZ, x, Ow, Ob, g1, b1, g2, b2):
    bn = 80
    grid = N // bn
    half = NP // bn
    row = pl.BlockSpec((bn, D), lambda i: (i, 0))
    row1 = pl.BlockSpec((bn, D), lambda i: (i + half, 0))
    w_spec = pl.BlockSpec((D, D), lambda i: (0, 0))
    b_spec = pl.BlockSpec((1, D), lambda i: (0, 0))
    return pl.pallas_call(
        _final_body,
        grid=(grid,),
        in_specs=[row, row1, row, row1, row,
                  w_spec, b_spec, b_spec, b_spec, b_spec, b_spec],
        out_specs=row,
        out_shape=jax.ShapeDtypeStruct((N, D), _f32),
    )(outV, outV, outZ, outZ, x, Ow, Ob, g1, b1, g2, b2)


def kernel(x, edge_index, Wq, bq, Wk, bk, Wv, bv, Ow, Ob,
           ln1_g, ln1_b, ln2_g, ln2_b):
    src = edge_index[0].astype(jnp.int32)
    dst = edge_index[1].astype(jnp.int32)
    bq2, bk2, bv2, ob2 = (b.reshape(1, D) for b in (bq, bk, bv, Ob))
    g1, b1, g2, b2 = (b.reshape(1, D) for b in (ln1_g, ln1_b, ln2_g, ln2_b))
    zeros = jnp.zeros((NP, D), _f32)

    Q, KV = _qkv(x, Wq, bq2, Wk, bk2, Wv, bv2)
    kvsrc, qdst = _sc_gather(Q, KV, src, dst)
    msgV, srep = _msg(kvsrc, qdst)
    outV = _sc_scatter(msgV, dst, zeros)
    outZ = _sc_scatter(srep, dst, zeros)
    return _final(outV, outZ, x, Ow, ob2, g1, b1, g2, b2)


# speedup vs baseline: 43.7282x; 1.4040x over previous
import jax
import jax.numpy as jnp
from jax import lax
from jax.experimental import pallas as pl
from jax.experimental.pallas import tpu as pltpu
from jax.experimental.pallas import tpu_sc as plsc

N = 10000
E = 320000
D = 128
H = 8
DH = 16

NC = 2
NS = 16
NW = NC * NS
PER_W = E // NW
C = 80
ITERS = PER_W // C
NP = 10240
STRIPE = NP // NS

_f32 = jnp.float32


def _qkv_body(x_ref, wq_ref, bq_ref, wk_ref, bk_ref, wv_ref, bv_ref,
              q_ref, kv_ref):
    x = x_ref[...]
    q = jnp.dot(x, wq_ref[...], preferred_element_type=_f32) + bq_ref[...]
    k = jnp.dot(x, wk_ref[...], preferred_element_type=_f32) + bk_ref[...]
    v = jnp.dot(x, wv_ref[...], preferred_element_type=_f32) + bv_ref[...]
    q_ref[...] = q
    k16 = lax.bitcast_convert_type(k.astype(jnp.bfloat16), jnp.uint16)
    v16 = lax.bitcast_convert_type(v.astype(jnp.bfloat16), jnp.uint16)
    kv_ref[...] = (v16.astype(jnp.int32) << 16) | k16.astype(jnp.int32)


def _qkv(x, Wq, bq, Wk, bk, Wv, bv):
    bn = 1000
    grid = N // bn
    row_spec = pl.BlockSpec((bn, D), lambda i: (i, 0))
    w_spec = pl.BlockSpec((D, D), lambda i: (0, 0))
    b_spec = pl.BlockSpec((1, D), lambda i: (0, 0))
    return pl.pallas_call(
        _qkv_body,
        grid=(grid,),
        in_specs=[row_spec, w_spec, b_spec, w_spec, b_spec, w_spec, b_spec],
        out_specs=[row_spec, row_spec],
        out_shape=[jax.ShapeDtypeStruct((N, D), _f32),
                   jax.ShapeDtypeStruct((N, D), jnp.int32)],
    )(x, Wq, bq, Wk, bk, Wv, bv)


def _gather_body(q_hbm, kv_hbm, src_hbm, dst_hbm,
                 kvsrc_out, qdst_out,
                 sidx0, sidx1, didx0, didx1, kvb0, kvb1, qb0, qb1,
                 idxs0, idxs1, gats0, gats1, wbs0, wbs1):
    wid = lax.axis_index("s") * NC + lax.axis_index("c")
    base = wid * PER_W
    slots = ((sidx0, didx0, kvb0, qb0, idxs0, gats0, wbs0),
             (sidx1, didx1, kvb1, qb1, idxs1, gats1, wbs1))

    pltpu.async_copy(src_hbm.at[pl.ds(base, C)], sidx0, idxs0)
    pltpu.async_copy(dst_hbm.at[pl.ds(base, C)], didx0, idxs0)

    def body(g, _):
        for b in (0, 1):
            sidx, didx, kvb, qb, idxs, gats, wbs = slots[b]
            osidx, odidx = slots[1 - b][0], slots[1 - b][1]
            oidxs = slots[1 - b][4]

            @pl.when((g & 1) == b)
            def _():
                off = pl.multiple_of(base + g * C, 8)

                @pl.when(g >= 2)
                def _():
                    pltpu.make_async_copy(kvb, kvsrc_out.at[pl.ds(base, C)],
                                          wbs).wait()
                    pltpu.make_async_copy(qb, qdst_out.at[pl.ds(base, C)],
                                          wbs).wait()

                pltpu.make_async_copy(src_hbm.at[pl.ds(base, C)], sidx,
                                      idxs).wait()
                pltpu.make_async_copy(dst_hbm.at[pl.ds(base, C)], didx,
                                      idxs).wait()

                cpk = pltpu.async_copy(kv_hbm.at[sidx], kvb, gats)
                cpq = pltpu.async_copy(q_hbm.at[didx], qb, gats)

                @pl.when(g + 1 < ITERS)
                def _():
                    off2 = pl.multiple_of(base + (g + 1) * C, 8)
                    pltpu.async_copy(src_hbm.at[pl.ds(off2, C)], osidx, oidxs)
                    pltpu.async_copy(dst_hbm.at[pl.ds(off2, C)], odidx, oidxs)

                cpk.wait()
                cpq.wait()
                pltpu.async_copy(kvb, kvsrc_out.at[pl.ds(off, C)], wbs)
                pltpu.async_copy(qb, qdst_out.at[pl.ds(off, C)], wbs)

        return 0

    lax.fori_loop(0, ITERS, body, 0)

    for b in (0, 1):
        _, _, kvb, qb, _, _, wbs = slots[b]
        pltpu.make_async_copy(kvb, kvsrc_out.at[pl.ds(base, C)], wbs).wait()
        pltpu.make_async_copy(qb, qdst_out.at[pl.ds(base, C)], wbs).wait()


def _sc_gather(Q, KV, src, dst):
    mesh = plsc.VectorSubcoreMesh(core_axis_name="c", subcore_axis_name="s")
    fn = pl.kernel(
        _gather_body,
        out_type=[jax.ShapeDtypeStruct((E, D), jnp.int32),
                  jax.ShapeDtypeStruct((E, D), _f32)],
        mesh=mesh,
        scratch_types=[
            pltpu.VMEM((C,), jnp.int32),
            pltpu.VMEM((C,), jnp.int32),
            pltpu.VMEM((C,), jnp.int32),
            pltpu.VMEM((C,), jnp.int32),
            pltpu.VMEM((C, D), jnp.int32),
            pltpu.VMEM((C, D), jnp.int32),
            pltpu.VMEM((C, D), _f32),
            pltpu.VMEM((C, D), _f32),
            pltpu.SemaphoreType.DMA,
            pltpu.SemaphoreType.DMA,
            pltpu.SemaphoreType.DMA,
            pltpu.SemaphoreType.DMA,
            pltpu.SemaphoreType.DMA,
            pltpu.SemaphoreType.DMA,
        ],
    )
    return fn(Q, KV, src, dst)


def _msg_body(kvsrc_ref, qdst_ref, msgv_ref, srep_ref):
    w = kvsrc_ref[...]
    ksrc = lax.bitcast_convert_type(w << 16, _f32)
    vsrc = lax.bitcast_convert_type(w & jnp.int32(-65536), _f32)
    prod = ksrc * qdst_ref[...]
    jr = lax.broadcasted_iota(jnp.int32, (D, H), 0) // DH
    jc = lax.broadcasted_iota(jnp.int32, (D, H), 1)
    s8 = (jr == jc).astype(_f32)
    score = jnp.dot(prod, s8, preferred_element_type=_f32)
    score = jnp.exp(jnp.clip(score * 0.25, -5.0, 5.0))
    hr = lax.broadcasted_iota(jnp.int32, (H, D), 0)
    hc = lax.broadcasted_iota(jnp.int32, (H, D), 1) // DH
    s128 = (hr == hc).astype(_f32)
    srep = jnp.dot(score, s128, preferred_element_type=_f32)
    msgv_ref[...] = vsrc * srep
    srep_ref[...] = srep


def _msg(kvsrc, qdst):
    be = 2000
    grid = E // be
    row_spec = pl.BlockSpec((be, D), lambda i: (i, 0))
    return pl.pallas_call(
        _msg_body,
        grid=(grid,),
        in_specs=[row_spec, row_spec],
        out_specs=[row_spec, row_spec],
        out_shape=[jax.ShapeDtypeStruct((E, D), _f32),
                   jax.ShapeDtypeStruct((E, D), _f32)],
    )(kvsrc, qdst)


def _scatter_body(msg_hbm, dst_hbm, zeros_hbm, out,
                  didx0, didx1, buf0, buf1, acc, lds0, lds1):
    c = lax.axis_index("c")
    s = lax.axis_index("s")
    wid = s * NC + c
    r0 = pl.multiple_of(s * STRIPE, 8)

    pltpu.sync_copy(zeros_hbm.at[pl.ds(r0, STRIPE)], acc.at[pl.ds(r0, STRIPE)])
    plsc.subcore_barrier()

    base = wid * PER_W
    slots = ((didx0, buf0, lds0), (didx1, buf1, lds1))

    pltpu.async_copy(dst_hbm.at[pl.ds(base, C)], didx0, lds0)
    pltpu.async_copy(msg_hbm.at[pl.ds(base, C)], buf0, lds0)

    def body(g, _):
        for b in (0, 1):
            didx, buf, lds = slots[b]
            odidx, obuf, olds = slots[1 - b]

            @pl.when((g & 1) == b)
            def _():
                pltpu.make_async_copy(dst_hbm.at[pl.ds(base, C)], didx,
                                      lds).wait()
                pltpu.make_async_copy(msg_hbm.at[pl.ds(base, C)], buf,
                                      lds).wait()

                @pl.when(g + 1 < ITERS)
                def _():
                    off2 = pl.multiple_of(base + (g + 1) * C, 8)
                    pltpu.async_copy(dst_hbm.at[pl.ds(off2, C)], odidx, olds)
                    pltpu.async_copy(msg_hbm.at[pl.ds(off2, C)], obuf, olds)

                pltpu.sync_copy(buf, acc.at[didx], add=True)

        return 0

    lax.fori_loop(0, ITERS, body, 0)
    plsc.subcore_barrier()

    o0 = pl.multiple_of(c * NP + r0, 8)
    pltpu.sync_copy(acc.at[pl.ds(r0, STRIPE)], out.at[pl.ds(o0, STRIPE)])


def _sc_scatter(msg, dst, zeros):
    mesh = plsc.VectorSubcoreMesh(core_axis_name="c", subcore_axis_name="s")
    fn = pl.kernel(
        _scatter_body,
        out_type=[jax.ShapeDtypeStruct((NC * NP, D), _f32)],
        mesh=mesh,
        scratch_types=[
            pltpu.VMEM((C,), jnp.int32),
            pltpu.VMEM((C,), jnp.int32),
            pltpu.VMEM((C, D), _f32),
            pltpu.VMEM((C, D), _f32),
            pltpu.VMEM_SHARED((NP, D), _f32),
            pltpu.SemaphoreType.DMA,
            pltpu.SemaphoreType.DMA,
        ],
    )
    (out,) = fn(msg, dst, zeros)
    return out


def _ln(h, g, b):
    m = jnp.mean(h, axis=-1, keepdims=True)
    v = jnp.mean((h - m) ** 2, axis=-1, keepdims=True)
    return (h - m) * lax.rsqrt(v + 1e-5) * g + b


def _final_body(pv0_ref, pv1_ref, pz0_ref, pz1_ref, x_ref,
                ow_ref, ob_ref, g1_ref, b1_ref, g2_ref, b2_ref, out_ref):
    wv = pv0_ref[...] + pv1_ref[...]
    zrep = pz0_ref[...] + pz1_ref[...]
    attn = wv / (zrep + 0.001)
    h1 = _ln(x_ref[...] + attn, g1_ref[...], b1_ref[...])
    h2 = _ln(h1, g2_ref[...], b2_ref[...])
    out_ref[...] = h1 + jax.nn.relu(
        jnp.dot(h2, ow_ref[...], preferred_element_type=_f32) + ob_ref[...])


def _final(outV, outZ, x, Ow, Ob, g1, b1, g2, b2):
    bn = 80
    grid = N // bn
    half = NP // bn
    row = pl.BlockSpec((bn, D), lambda i: (i, 0))
    row1 = pl.BlockSpec((bn, D), lambda i: (i + half, 0))
    w_spec = pl.BlockSpec((D, D), lambda i: (0, 0))
    b_spec = pl.BlockSpec((1, D), lambda i: (0, 0))
    return pl.pallas_call(
        _final_body,
        grid=(grid,),
        in_specs=[row, row1, row, row1, row,
                  w_spec, b_spec, b_spec, b_spec, b_spec, b_spec],
        out_specs=row,
        out_shape=jax.ShapeDtypeStruct((N, D), _f32),
    )(outV, outV, outZ, outZ, x, Ow, Ob, g1, b1, g2, b2)


def kernel(x, edge_index, Wq, bq, Wk, bk, Wv, bv, Ow, Ob,
           ln1_g, ln1_b, ln2_g, ln2_b):
    src = edge_index[0].astype(jnp.int32)
    dst = edge_index[1].astype(jnp.int32)
    bq2, bk2, bv2, ob2 = (b.reshape(1, D) for b in (bq, bk, bv, Ob))
    g1, b1, g2, b2 = (b.reshape(1, D) for b in (ln1_g, ln1_b, ln2_g, ln2_b))
    zeros = jnp.zeros((NP, D), _f32)

    Q, KV = _qkv(x, Wq, bq2, Wk, bk2, Wv, bv2)
    kvsrc, qdst = _sc_gather(Q, KV, src, dst)
    msgV, srep = _msg(kvsrc, qdst)
    outV = _sc_scatter(msgV, dst, zeros)
    outZ = _sc_scatter(srep, dst, zeros)
    return _final(outV, outZ, x, Ow, ob2, g1, b1, g2, b2)
